# Initial kernel scaffold; baseline (speedup 1.0000x reference)
import functools

import jax
import jax.numpy as jnp
from jax import lax
from jax.experimental import pallas as pl
from jax.experimental.pallas import tpu as pltpu
from jax.experimental.pallas import tpu_sc as plsc

_LANES = 16


def _make_sc_kernel(num_nodes, feat, num_edges):
    info = plsc.get_sparse_core_info()
    nc, ns = info.num_cores, info.num_subcores
    nw = nc * ns
    assert num_edges % nw == 0
    e_per_w = num_edges // nw

    chunk = 400
    assert e_per_w % chunk == 0 and chunk % _LANES == 0
    n_chunks = e_per_w // chunk
    n_groups = chunk // _LANES
    j_steps = feat // _LANES

    mesh = plsc.VectorSubcoreMesh(core_axis_name="c", subcore_axis_name="s")

    @functools.partial(
        pl.kernel,
        mesh=mesh,
        out_type=jax.ShapeDtypeStruct((num_edges,), jnp.float32),
        scratch_types=[
            pltpu.VMEM((chunk,), jnp.int32),
            pltpu.VMEM((chunk,), jnp.int32),
            pltpu.VMEM((chunk, feat), jnp.float32),
            pltpu.VMEM((chunk, feat), jnp.float32),
            pltpu.VMEM((chunk,), jnp.float32),
            pltpu.SemaphoreType.DMA,
            pltpu.SemaphoreType.DMA,
        ],
    )
    def kern(z_hbm, src_hbm, dst_hbm, out_hbm,
             idx_s, idx_d, rows_s, rows_d, out_v, sem_s, sem_d):
        wid = lax.axis_index("s") * nc + lax.axis_index("c")
        wbase = wid * e_per_w
        lane_iota = lax.iota(jnp.int32, _LANES)

        def chunk_body(c, _):
            base = wbase + c * chunk
            pltpu.sync_copy(src_hbm.at[pl.ds(base, chunk)], idx_s)
            pltpu.sync_copy(dst_hbm.at[pl.ds(base, chunk)], idx_d)
            cp_s = pltpu.async_copy(z_hbm.at[idx_s], rows_s, sem_s)
            cp_d = pltpu.async_copy(z_hbm.at[idx_d], rows_d, sem_d)
            cp_s.wait()
            cp_d.wait()

            def group_body(g, _):
                res = jnp.zeros((_LANES,), jnp.float32)
                for lane in range(_LANES):
                    e = g * _LANES + lane
                    acc = rows_s[e, pl.ds(0, _LANES)] * rows_d[e, pl.ds(0, _LANES)]
                    for j in range(1, j_steps):
                        acc = acc + (rows_s[e, pl.ds(j * _LANES, _LANES)]
                                     * rows_d[e, pl.ds(j * _LANES, _LANES)])
                    dot = jnp.sum(acc)
                    res = jnp.where(lane_iota == lane, dot, res)
                out_v[pl.ds(g * _LANES, _LANES)] = res
                return 0

            lax.fori_loop(0, n_groups, group_body, 0)
            pltpu.sync_copy(out_v, out_hbm.at[pl.ds(base, chunk)])
            return 0

        lax.fori_loop(0, n_chunks, chunk_body, 0)

    return kern


def kernel(z, edge_index):
    num_nodes, feat = z.shape
    num_edges = edge_index.shape[1]
    kern = _make_sc_kernel(num_nodes, feat, num_edges)
    src = edge_index[0]
    dst = edge_index[1]
    return kern(z, src, dst)


# speedup vs baseline: 1.1997x; 1.1997x over previous
import functools

import jax
import jax.numpy as jnp
from jax import lax
from jax.experimental import pallas as pl
from jax.experimental.pallas import tpu as pltpu
from jax.experimental.pallas import tpu_sc as plsc

_LANES = 16


def _make_sc_kernel(num_nodes, feat, num_edges):
    info = plsc.get_sparse_core_info()
    nc, ns = info.num_cores, info.num_subcores
    nw = nc * ns
    assert num_edges % nw == 0
    e_per_w = num_edges // nw

    chunk = 400
    assert e_per_w % chunk == 0 and chunk % _LANES == 0
    n_chunks = e_per_w // chunk
    n_groups = chunk // _LANES
    j_steps = feat // _LANES

    mesh = plsc.VectorSubcoreMesh(core_axis_name="c", subcore_axis_name="s")

    @functools.partial(
        pl.kernel,
        mesh=mesh,
        out_type=jax.ShapeDtypeStruct((num_edges,), jnp.float32),
        scratch_types=[
            pltpu.VMEM((chunk,), jnp.int32),
            pltpu.VMEM((chunk,), jnp.int32),
            pltpu.VMEM((chunk, feat), jnp.float32),
            pltpu.VMEM((chunk, feat), jnp.float32),
            pltpu.VMEM((chunk,), jnp.float32),
            pltpu.SemaphoreType.DMA,
            pltpu.SemaphoreType.DMA,
        ],
        compiler_params=pltpu.CompilerParams(needs_layout_passes=False),
    )
    def kern(z_hbm, src_hbm, dst_hbm, out_hbm,
             idx_s, idx_d, rows_s, rows_d, out_v, sem_s, sem_d):
        wid = lax.axis_index("s") * nc + lax.axis_index("c")
        wbase = wid * e_per_w
        lane_iota = lax.iota(jnp.int32, _LANES)

        def chunk_body(c, _):
            base = wbase + c * chunk
            pltpu.sync_copy(src_hbm.at[pl.ds(base, chunk)], idx_s)
            pltpu.sync_copy(dst_hbm.at[pl.ds(base, chunk)], idx_d)
            cp_s = pltpu.async_copy(z_hbm.at[idx_s], rows_s, sem_s)
            cp_d = pltpu.async_copy(z_hbm.at[idx_d], rows_d, sem_d)
            cp_s.wait()
            cp_d.wait()

            def group_body(g, _):
                row_idx = g * _LANES + lane_iota
                col0 = jnp.zeros((_LANES,), jnp.int32)
                acc = (plsc.load_gather(rows_s, [row_idx, col0])
                       * plsc.load_gather(rows_d, [row_idx, col0]))
                for d in range(1, feat):
                    col = col0 + d
                    acc = acc + (plsc.load_gather(rows_s, [row_idx, col])
                                 * plsc.load_gather(rows_d, [row_idx, col]))
                out_v[pl.ds(g * _LANES, _LANES)] = acc
                return 0

            lax.fori_loop(0, n_groups, group_body, 0)
            pltpu.sync_copy(out_v, out_hbm.at[pl.ds(base, chunk)])
            return 0

        lax.fori_loop(0, n_chunks, chunk_body, 0)

    return kern


def kernel(z, edge_index):
    num_nodes, feat = z.shape
    num_edges = edge_index.shape[1]
    kern = _make_sc_kernel(num_nodes, feat, num_edges)
    src = edge_index[0]
    dst = edge_index[1]
    return kern(z, src, dst)


# speedup vs baseline: 4.8099x; 4.0091x over previous
import functools

import jax
import jax.numpy as jnp
from jax import lax
from jax.experimental import pallas as pl
from jax.experimental.pallas import tpu as pltpu
from jax.experimental.pallas import tpu_sc as plsc

_LANES = 16


def _make_sc_kernel(num_nodes, feat, num_edges):
    info = plsc.get_sparse_core_info()
    nc, ns = info.num_cores, info.num_subcores
    nw = nc * ns
    assert num_edges % nw == 0
    e_per_w = num_edges // nw

    chunk = 400
    assert e_per_w % chunk == 0 and chunk % _LANES == 0
    n_chunks = e_per_w // chunk
    n_groups = chunk // _LANES
    assert feat % _LANES == 0 and feat & (feat - 1) == 0

    mesh = plsc.VectorSubcoreMesh(core_axis_name="c", subcore_axis_name="s")

    @functools.partial(
        pl.kernel,
        mesh=mesh,
        out_type=jax.ShapeDtypeStruct((num_edges,), jnp.float32),
        scratch_types=[
            pltpu.VMEM((chunk,), jnp.int32),
            pltpu.VMEM((chunk,), jnp.int32),
            pltpu.VMEM((chunk, feat), jnp.float32),
            pltpu.VMEM((chunk, feat), jnp.float32),
            pltpu.VMEM((chunk,), jnp.float32),
            pltpu.SemaphoreType.DMA,
            pltpu.SemaphoreType.DMA,
        ],
        compiler_params=pltpu.CompilerParams(needs_layout_passes=False),
    )
    def kern(z_hbm, src_hbm, dst_hbm, out_hbm,
             idx_s, idx_d, rows_s, rows_d, out_v, sem_s, sem_d):
        wid = lax.axis_index("s") * nc + lax.axis_index("c")
        wbase = wid * e_per_w
        lane_iota = lax.iota(jnp.int32, _LANES)

        def chunk_body(c, _):
            base = wbase + c * chunk
            pltpu.sync_copy(src_hbm.at[pl.ds(base, chunk)], idx_s)
            pltpu.sync_copy(dst_hbm.at[pl.ds(base, chunk)], idx_d)
            cp_s = pltpu.async_copy(z_hbm.at[idx_s], rows_s, sem_s)
            cp_d = pltpu.async_copy(z_hbm.at[idx_d], rows_d, sem_d)
            cp_s.wait()
            cp_d.wait()

            def group_body(g, _):
                row_idx = g * _LANES + lane_iota
                col = lane_iota
                acc = (plsc.load_gather(rows_s, [row_idx, col])
                       * plsc.load_gather(rows_d, [row_idx, col]))
                for _ in range(1, feat):
                    col = (col + 1) & (feat - 1)
                    acc = acc + (plsc.load_gather(rows_s, [row_idx, col])
                                 * plsc.load_gather(rows_d, [row_idx, col]))
                out_v[pl.ds(g * _LANES, _LANES)] = acc
                return 0

            lax.fori_loop(0, n_groups, group_body, 0)
            pltpu.sync_copy(out_v, out_hbm.at[pl.ds(base, chunk)])
            return 0

        lax.fori_loop(0, n_chunks, chunk_body, 0)

    return kern


def kernel(z, edge_index):
    num_nodes, feat = z.shape
    num_edges = edge_index.shape[1]
    kern = _make_sc_kernel(num_nodes, feat, num_edges)
    src = edge_index[0]
    dst = edge_index[1]
    return kern(z, src, dst)


# speedup vs baseline: 7.9175x; 1.6461x over previous
import functools

import jax
import jax.numpy as jnp
from jax import lax
from jax.experimental import pallas as pl
from jax.experimental.pallas import tpu as pltpu
from jax.experimental.pallas import tpu_sc as plsc

_LANES = 16


def _make_sc_kernel(num_nodes, feat, num_edges):
    info = plsc.get_sparse_core_info()
    nc, ns = info.num_cores, info.num_subcores
    nw = nc * ns
    assert num_edges % nw == 0
    e_per_w = num_edges // nw

    chunk = 80
    assert e_per_w % chunk == 0 and chunk % _LANES == 0
    n_chunks = e_per_w // chunk
    n_groups = chunk // _LANES
    assert feat % _LANES == 0 and feat & (feat - 1) == 0

    mesh = plsc.VectorSubcoreMesh(core_axis_name="c", subcore_axis_name="s")

    @functools.partial(
        pl.kernel,
        mesh=mesh,
        out_type=jax.ShapeDtypeStruct((num_edges,), jnp.float32),
        scratch_types=[
            pltpu.VMEM((e_per_w,), jnp.int32),
            pltpu.VMEM((e_per_w,), jnp.int32),
            pltpu.VMEM((2 * chunk, feat), jnp.float32),
            pltpu.VMEM((2 * chunk, feat), jnp.float32),
            pltpu.VMEM((e_per_w,), jnp.float32),
            pltpu.SemaphoreType.DMA,
            pltpu.SemaphoreType.DMA,
        ],
        compiler_params=pltpu.CompilerParams(needs_layout_passes=False),
    )
    def kern(z_hbm, src_hbm, dst_hbm, out_hbm,
             idx_s, idx_d, rows_s, rows_d, out_v, sem_s, sem_d):
        wid = lax.axis_index("s") * nc + lax.axis_index("c")
        wbase = wid * e_per_w
        lane_iota = lax.iota(jnp.int32, _LANES)

        pltpu.sync_copy(src_hbm.at[pl.ds(wbase, e_per_w)], idx_s)
        pltpu.sync_copy(dst_hbm.at[pl.ds(wbase, e_per_w)], idx_d)

        def gather_start(c, buf):
            pltpu.async_copy(
                z_hbm.at[idx_s.at[pl.ds(c * chunk, chunk)]],
                rows_s.at[pl.ds(buf * chunk, chunk)], sem_s)
            pltpu.async_copy(
                z_hbm.at[idx_d.at[pl.ds(c * chunk, chunk)]],
                rows_d.at[pl.ds(buf * chunk, chunk)], sem_d)

        def gather_wait(c, buf):
            pltpu.make_async_copy(
                z_hbm.at[idx_s.at[pl.ds(c * chunk, chunk)]],
                rows_s.at[pl.ds(buf * chunk, chunk)], sem_s).wait()
            pltpu.make_async_copy(
                z_hbm.at[idx_d.at[pl.ds(c * chunk, chunk)]],
                rows_d.at[pl.ds(buf * chunk, chunk)], sem_d).wait()

        gather_start(0, 0)

        def chunk_body(c, _):
            buf = lax.rem(c, 2)
            gather_wait(c, buf)

            @pl.when(c + 1 < n_chunks)
            def _():
                gather_start(c + 1, 1 - buf)

            rbase = buf * chunk

            def group_body(g, _):
                row_idx = rbase + g * _LANES + lane_iota
                col = lane_iota
                acc = (plsc.load_gather(rows_s, [row_idx, col])
                       * plsc.load_gather(rows_d, [row_idx, col]))
                for _ in range(1, feat):
                    col = (col + 1) & (feat - 1)
                    acc = acc + (plsc.load_gather(rows_s, [row_idx, col])
                                 * plsc.load_gather(rows_d, [row_idx, col]))
                out_v[pl.ds(c * chunk + g * _LANES, _LANES)] = acc
                return 0

            lax.fori_loop(0, n_groups, group_body, 0)
            return 0

        lax.fori_loop(0, n_chunks, chunk_body, 0)
        pltpu.sync_copy(out_v, out_hbm.at[pl.ds(wbase, e_per_w)])

    return kern


def kernel(z, edge_index):
    num_nodes, feat = z.shape
    num_edges = edge_index.shape[1]
    kern = _make_sc_kernel(num_nodes, feat, num_edges)
    src = edge_index[0]
    dst = edge_index[1]
    return kern(z, src, dst)
